# baseline (speedup 1.0000x reference)
import functools

import jax
import jax.numpy as jnp
from jax import lax
from jax.experimental import pallas as pl
from jax.experimental.pallas import tpu as pltpu
from jax.experimental.pallas import tpu_sc as plsc

S = 2048
H = 2048
F = 4096
E = 8
K = 2

T = 1024
MT = (S * K) // T + E
PAD_N = MT * T
FN = 512
NF = F // FN

NC, NS, NL = 2, 16, 16
NW = NC * NS
TOK_W = S // NW
CHT = 16
NCH = TOK_W // CHT


def _router_body(x_ref, gw_ref, logits_ref, sel_ref, topw_ref, pos_ref,
                 te_ref, w0x_ref, w1x_ref):
    x = x_ref[...]
    gw = gw_ref[...]
    logits = lax.dot_general(x, gw, (((1,), (1,)), ((), ())),
                             preferred_element_type=jnp.float32)
    logits_ref[...] = logits

    m = jnp.max(logits, axis=1, keepdims=True)
    p = jnp.exp(logits - m)
    rw = p / jnp.sum(p, axis=1, keepdims=True)

    iota_e = lax.broadcasted_iota(jnp.int32, (S, E), 1)
    w1 = jnp.max(rw, axis=1, keepdims=True)
    i1 = jnp.min(jnp.where(rw == w1, iota_e, E), axis=1, keepdims=True)
    rw2 = jnp.where(iota_e == i1, -jnp.inf, rw)
    w2 = jnp.max(rw2, axis=1, keepdims=True)
    i2 = jnp.min(jnp.where(rw2 == w2, iota_e, E), axis=1, keepdims=True)
    ssum = w1 + w2
    sel_ref[...] = jnp.concatenate([i1, i2], axis=1)
    topw_ref[...] = jnp.concatenate([w1 / ssum, w2 / ssum], axis=1)
    w0x_ref[...] = jnp.broadcast_to(w1 / ssum, (S, NL))
    w1x_ref[...] = jnp.broadcast_to(w2 / ssum, (S, NL))

    oh0 = (iota_e == i1).astype(jnp.float32)
    oh1 = (iota_e == i2).astype(jnp.float32)
    ohsum = oh0 + oh1
    CH = 256
    tri = (lax.broadcasted_iota(jnp.int32, (CH, CH), 0)
           > lax.broadcasted_iota(jnp.int32, (CH, CH), 1)).astype(jnp.float32)
    chunks = []
    carry = jnp.zeros((1, E), jnp.float32)
    for c in range(S // CH):
        blk = ohsum[c * CH:(c + 1) * CH, :]
        pref = lax.dot_general(tri, blk, (((1,), (0,)), ((), ())),
                               preferred_element_type=jnp.float32)
        chunks.append(pref + carry)
        carry = carry + jnp.sum(blk, axis=0, keepdims=True)
    P = jnp.concatenate(chunks, axis=0)
    counts = carry

    rank0 = jnp.sum(P * oh0, axis=1, keepdims=True)
    rank1 = jnp.sum(P * oh1, axis=1, keepdims=True)

    cap = jnp.ceil(counts / T) * T
    upper = (lax.broadcasted_iota(jnp.int32, (E, E), 0)
             < lax.broadcasted_iota(jnp.int32, (E, E), 1)).astype(jnp.float32)
    start = lax.dot_general(cap, upper, (((1,), (0,)), ((), ())),
                            preferred_element_type=jnp.float32)
    ends = start + cap

    pos0 = rank0 + jnp.sum(oh0 * start, axis=1, keepdims=True)
    pos1 = rank1 + jnp.sum(oh1 * start, axis=1, keepdims=True)
    pos_ref[...] = jnp.concatenate([pos0, pos1], axis=1).astype(jnp.int32)

    tile_base = lax.broadcasted_iota(jnp.int32, (1, 128), 1) * T
    te = jnp.zeros((1, 128), jnp.int32)
    for e in range(E):
        end_e = ends[0, e].astype(jnp.int32)
        te = te + (tile_base >= end_e).astype(jnp.int32)
    te = jnp.minimum(te, E - 1)
    used = (ends[0, E - 1] / T).astype(jnp.int32)
    lane = lax.broadcasted_iota(jnp.int32, (1, 128), 1)
    te_ref[...] = jnp.where(lane == 120, used, te)


def _run_router(x, gate_w):
    return pl.pallas_call(
        _router_body,
        out_shape=[
            jax.ShapeDtypeStruct((S, E), jnp.float32),
            jax.ShapeDtypeStruct((S, K), jnp.int32),
            jax.ShapeDtypeStruct((S, K), jnp.float32),
            jax.ShapeDtypeStruct((S, K), jnp.int32),
            jax.ShapeDtypeStruct((1, 128), jnp.int32),
            jax.ShapeDtypeStruct((S, NL), jnp.float32),
            jax.ShapeDtypeStruct((S, NL), jnp.float32),
        ],
    )(x, gate_w)


@functools.cache
def _make_scatter_rows():
    mesh = plsc.VectorSubcoreMesh(core_axis_name="c", subcore_axis_name="s")

    @functools.partial(
        pl.kernel,
        out_type=jax.ShapeDtypeStruct((PAD_N, H // 2), jnp.int32),
        mesh=mesh,
        scratch_types=[
            pltpu.VMEM((CHT, H // 2), jnp.int32),
            pltpu.VMEM((CHT,), jnp.int32),
            pltpu.VMEM((CHT,), jnp.int32),
            pltpu.SemaphoreType.DMA,
        ],
    )
    def scatter_rows(x_hbm, pos0_hbm, pos1_hbm, xs_hbm, rows_v, i0_v, i1_v,
                     sem):
        wid = lax.axis_index("s") * NC + lax.axis_index("c")
        for j in range(NCH):
            tb = wid * TOK_W + j * CHT
            pltpu.sync_copy(x_hbm.at[pl.ds(tb, CHT)], rows_v)
            pltpu.sync_copy(pos0_hbm.at[pl.ds(tb, CHT)], i0_v)
            pltpu.sync_copy(pos1_hbm.at[pl.ds(tb, CHT)], i1_v)
            pltpu.async_copy(rows_v, xs_hbm.at[i0_v], sem).wait()
            pltpu.async_copy(rows_v, xs_hbm.at[i1_v], sem).wait()

    return scatter_rows


def _mlp_body(te_ref, xs_ref, g_ref, u_ref, d_ref, o_ref):
    mt = pl.program_id(0)
    nf = pl.program_id(1)

    @pl.when(mt < te_ref[120])
    def _():
        xi = xs_ref[...]
        xa = lax.bitcast_convert_type(
            jnp.bitwise_and(xi, jnp.int32(-65536)), jnp.float32)
        xb = lax.bitcast_convert_type(jnp.left_shift(xi, 16), jnp.float32)
        x = jnp.concatenate([xa, xb], axis=1).astype(jnp.bfloat16)
        g = lax.dot_general(x, g_ref[0].astype(jnp.bfloat16),
                            (((1,), (1,)), ((), ())),
                            preferred_element_type=jnp.float32)
        u = lax.dot_general(x, u_ref[0].astype(jnp.bfloat16),
                            (((1,), (1,)), ((), ())),
                            preferred_element_type=jnp.float32)
        act = (g * jax.nn.sigmoid(g) * u).astype(jnp.bfloat16)
        y = lax.dot_general(act, d_ref[0].astype(jnp.bfloat16),
                            (((1,), (1,)), ((), ())),
                            preferred_element_type=jnp.float32)

        @pl.when(nf == 0)
        def _():
            o_ref[...] = y

        @pl.when(nf > 0)
        def _():
            o_ref[...] += y


def _run_mlp(te, xs, gate_proj, up_proj, down_proj):
    def _wmap(mt, nf, te, dim):
        cond = mt < te[120]
        return (jnp.where(cond, te[mt], E - 1),
                jnp.where(cond, nf, NF - 1), 0)[dim]

    return pl.pallas_call(
        _mlp_body,
        grid_spec=pltpu.PrefetchScalarGridSpec(
            num_scalar_prefetch=1,
            grid=(MT, NF),
            in_specs=[
                pl.BlockSpec(
                    (T, H // 2),
                    lambda mt, nf, te: (jnp.where(mt < te[120], mt,
                                                  te[120] - 1), 0)),
                pl.BlockSpec(
                    (1, FN, H),
                    lambda mt, nf, te: (_wmap(mt, nf, te, 0),
                                        _wmap(mt, nf, te, 1), 0)),
                pl.BlockSpec(
                    (1, FN, H),
                    lambda mt, nf, te: (_wmap(mt, nf, te, 0),
                                        _wmap(mt, nf, te, 1), 0)),
                pl.BlockSpec(
                    (1, H, FN),
                    lambda mt, nf, te: (_wmap(mt, nf, te, 0), 0,
                                        _wmap(mt, nf, te, 1))),
            ],
            out_specs=pl.BlockSpec(
                (T, H),
                lambda mt, nf, te: (jnp.where(mt < te[120], mt,
                                              te[120] - 1), 0)),
        ),
        out_shape=jax.ShapeDtypeStruct((PAD_N, H), jnp.float32),
    )(te, xs, gate_proj, up_proj, down_proj)


@functools.cache
def _make_combine():
    mesh = plsc.VectorSubcoreMesh(core_axis_name="c", subcore_axis_name="s")

    @functools.partial(
        pl.kernel,
        out_type=jax.ShapeDtypeStruct((S, H), jnp.float32),
        mesh=mesh,
        scratch_types=[
            pltpu.VMEM((CHT, H), jnp.float32),
            pltpu.VMEM((CHT, H), jnp.float32),
            pltpu.VMEM((CHT,), jnp.int32),
            pltpu.VMEM((CHT,), jnp.int32),
            pltpu.VMEM((CHT, NL), jnp.float32),
            pltpu.VMEM((CHT, NL), jnp.float32),
            pltpu.SemaphoreType.DMA,
        ],
    )
    def combine(ys_hbm, pos0_hbm, pos1_hbm, w0_hbm, w1_hbm, out_hbm,
                r0_v, r1_v, i0_v, i1_v, w0_v, w1_v, sem):
        wid = lax.axis_index("s") * NC + lax.axis_index("c")
        for j in range(NCH):
            tb = wid * TOK_W + j * CHT
            pltpu.sync_copy(pos0_hbm.at[pl.ds(tb, CHT)], i0_v)
            pltpu.sync_copy(pos1_hbm.at[pl.ds(tb, CHT)], i1_v)
            pltpu.sync_copy(w0_hbm.at[pl.ds(tb, CHT)], w0_v)
            pltpu.sync_copy(w1_hbm.at[pl.ds(tb, CHT)], w1_v)
            pltpu.async_copy(ys_hbm.at[i0_v], r0_v, sem).wait()
            pltpu.async_copy(ys_hbm.at[i1_v], r1_v, sem).wait()
            for i in range(CHT):
                w0s = w0_v[i, :]
                w1s = w1_v[i, :]

                def _col(cc, _, i=i, w0s=w0s, w1s=w1s):
                    sl = pl.ds(cc * NL, NL)
                    r0_v[i, sl] = w0s * r0_v[i, sl] + w1s * r1_v[i, sl]
                    return 0

                lax.fori_loop(0, H // NL, _col, 0)
            pltpu.sync_copy(r0_v, out_hbm.at[pl.ds(tb, CHT)])

    return combine


def kernel(hidden_states, gate_w, gate_proj, up_proj, down_proj):
    b, s, h = hidden_states.shape
    x = hidden_states.reshape(s, h)

    logits, sel, topw, pos, te128, w0x, w1x = _run_router(x, gate_w)

    pos0 = pos[:, 0]
    pos1 = pos[:, 1]
    te = te128.reshape(128)

    hi = lax.bitcast_convert_type(x[:, :h // 2].astype(jnp.bfloat16),
                                  jnp.uint16).astype(jnp.int32)
    lo = lax.bitcast_convert_type(x[:, h // 2:].astype(jnp.bfloat16),
                                  jnp.uint16).astype(jnp.int32)
    xi = jnp.bitwise_or(jnp.left_shift(hi, 16), lo)
    xsi = _make_scatter_rows()(xi, pos0, pos1)
    ys = _run_mlp(te, xsi, gate_proj, up_proj, down_proj)
    final = _make_combine()(ys, pos0, pos1, w0x, w1x)

    return (final.reshape(b, s, h), sel.reshape(b, s, K),
            topw.reshape(b, s, K), logits.reshape(b, s, E))


# speedup vs baseline: 1.0684x; 1.0684x over previous
import functools

import jax
import jax.numpy as jnp
from jax import lax
from jax.experimental import pallas as pl
from jax.experimental.pallas import tpu as pltpu
from jax.experimental.pallas import tpu_sc as plsc

S = 2048
H = 2048
F = 4096
E = 8
K = 2

T = 512
MT = (S * K) // T + E
PAD_N = MT * T
FN = 512
NF = F // FN

NC, NS, NL = 2, 16, 16
NW = NC * NS
TOK_W = S // NW
CHT = 16
WL = 128
NCH = TOK_W // CHT


def _router_body(x_ref, gw_ref, logits_ref, sel_ref, topw_ref, pos_ref,
                 te_ref, w0x_ref, w1x_ref):
    x = x_ref[...]
    gw = gw_ref[...]
    logits = lax.dot_general(x, gw, (((1,), (1,)), ((), ())),
                             preferred_element_type=jnp.float32)
    logits_ref[...] = logits

    m = jnp.max(logits, axis=1, keepdims=True)
    p = jnp.exp(logits - m)
    rw = p / jnp.sum(p, axis=1, keepdims=True)

    iota_e = lax.broadcasted_iota(jnp.int32, (S, E), 1)
    w1 = jnp.max(rw, axis=1, keepdims=True)
    i1 = jnp.min(jnp.where(rw == w1, iota_e, E), axis=1, keepdims=True)
    rw2 = jnp.where(iota_e == i1, -jnp.inf, rw)
    w2 = jnp.max(rw2, axis=1, keepdims=True)
    i2 = jnp.min(jnp.where(rw2 == w2, iota_e, E), axis=1, keepdims=True)
    ssum = w1 + w2
    sel_ref[...] = jnp.concatenate([i1, i2], axis=1)
    topw_ref[...] = jnp.concatenate([w1 / ssum, w2 / ssum], axis=1)
    w0x_ref[...] = jnp.broadcast_to(w1 / ssum, (S, WL))
    w1x_ref[...] = jnp.broadcast_to(w2 / ssum, (S, WL))

    oh0 = (iota_e == i1).astype(jnp.float32)
    oh1 = (iota_e == i2).astype(jnp.float32)
    ohsum = oh0 + oh1
    CH = 256
    tri = (lax.broadcasted_iota(jnp.int32, (CH, CH), 0)
           > lax.broadcasted_iota(jnp.int32, (CH, CH), 1)).astype(jnp.float32)
    chunks = []
    carry = jnp.zeros((1, E), jnp.float32)
    for c in range(S // CH):
        blk = ohsum[c * CH:(c + 1) * CH, :]
        pref = lax.dot_general(tri, blk, (((1,), (0,)), ((), ())),
                               preferred_element_type=jnp.float32)
        chunks.append(pref + carry)
        carry = carry + jnp.sum(blk, axis=0, keepdims=True)
    P = jnp.concatenate(chunks, axis=0)
    counts = carry

    rank0 = jnp.sum(P * oh0, axis=1, keepdims=True)
    rank1 = jnp.sum(P * oh1, axis=1, keepdims=True)

    cap = jnp.ceil(counts / T) * T
    upper = (lax.broadcasted_iota(jnp.int32, (E, E), 0)
             < lax.broadcasted_iota(jnp.int32, (E, E), 1)).astype(jnp.float32)
    start = lax.dot_general(cap, upper, (((1,), (0,)), ((), ())),
                            preferred_element_type=jnp.float32)
    ends = start + cap

    pos0 = rank0 + jnp.sum(oh0 * start, axis=1, keepdims=True)
    pos1 = rank1 + jnp.sum(oh1 * start, axis=1, keepdims=True)
    pos_ref[...] = jnp.concatenate([pos0, pos1], axis=1).astype(jnp.int32)

    tile_base = lax.broadcasted_iota(jnp.int32, (1, 128), 1) * T
    te = jnp.zeros((1, 128), jnp.int32)
    for e in range(E):
        end_e = ends[0, e].astype(jnp.int32)
        te = te + (tile_base >= end_e).astype(jnp.int32)
    te = jnp.minimum(te, E - 1)
    used = (ends[0, E - 1] / T).astype(jnp.int32)
    lane = lax.broadcasted_iota(jnp.int32, (1, 128), 1)
    te_ref[...] = jnp.where(lane == 120, used, te)


def _run_router(x, gate_w):
    return pl.pallas_call(
        _router_body,
        out_shape=[
            jax.ShapeDtypeStruct((S, E), jnp.float32),
            jax.ShapeDtypeStruct((S, K), jnp.int32),
            jax.ShapeDtypeStruct((S, K), jnp.float32),
            jax.ShapeDtypeStruct((S, K), jnp.int32),
            jax.ShapeDtypeStruct((1, 128), jnp.int32),
            jax.ShapeDtypeStruct((S, WL), jnp.float32),
            jax.ShapeDtypeStruct((S, WL), jnp.float32),
        ],
    )(x, gate_w)


@functools.cache
def _make_scatter_rows():
    mesh = plsc.VectorSubcoreMesh(core_axis_name="c", subcore_axis_name="s")

    @functools.partial(
        pl.kernel,
        out_type=[
            jax.ShapeDtypeStruct((PAD_N, H // 2), jnp.int32),
            jax.ShapeDtypeStruct((PAD_N, WL), jnp.float32),
        ],
        mesh=mesh,
        scratch_types=[
            pltpu.VMEM((CHT, H // 2), jnp.int32),
            pltpu.VMEM((CHT,), jnp.int32),
            pltpu.VMEM((CHT,), jnp.int32),
            pltpu.VMEM((CHT, WL), jnp.float32),
            pltpu.VMEM((CHT, WL), jnp.float32),
            pltpu.SemaphoreType.DMA,
        ],
    )
    def scatter_rows(x_hbm, pos0_hbm, pos1_hbm, w0x_hbm, w1x_hbm,
                     xs_hbm, wrow_hbm, rows_v, i0_v, i1_v, w0m_v, w1m_v,
                     sem):
        wid = lax.axis_index("s") * NC + lax.axis_index("c")
        for j in range(NCH):
            tb = wid * TOK_W + j * CHT
            pltpu.sync_copy(x_hbm.at[pl.ds(tb, CHT)], rows_v)
            pltpu.sync_copy(pos0_hbm.at[pl.ds(tb, CHT)], i0_v)
            pltpu.sync_copy(pos1_hbm.at[pl.ds(tb, CHT)], i1_v)
            pltpu.sync_copy(w0x_hbm.at[pl.ds(tb, CHT)], w0m_v)
            pltpu.sync_copy(w1x_hbm.at[pl.ds(tb, CHT)], w1m_v)
            pltpu.async_copy(rows_v, xs_hbm.at[i0_v], sem).wait()
            pltpu.async_copy(rows_v, xs_hbm.at[i1_v], sem).wait()
            pltpu.async_copy(w0m_v, wrow_hbm.at[i0_v], sem).wait()
            pltpu.async_copy(w1m_v, wrow_hbm.at[i1_v], sem).wait()

    return scatter_rows


def _mlp_body(te_ref, xs_ref, g_ref, u_ref, d_ref, wr_ref, o_ref):
    mt = pl.program_id(0)
    nf = pl.program_id(1)

    @pl.when(mt < te_ref[120])
    def _():
        xi = xs_ref[...]
        xa = lax.bitcast_convert_type(
            jnp.bitwise_and(xi, jnp.int32(-65536)), jnp.float32)
        xb = lax.bitcast_convert_type(jnp.left_shift(xi, 16), jnp.float32)
        x = jnp.concatenate([xa, xb], axis=1).astype(jnp.bfloat16)
        g = lax.dot_general(x, g_ref[0].astype(jnp.bfloat16),
                            (((1,), (1,)), ((), ())),
                            preferred_element_type=jnp.float32)
        u = lax.dot_general(x, u_ref[0].astype(jnp.bfloat16),
                            (((1,), (1,)), ((), ())),
                            preferred_element_type=jnp.float32)
        act = (g * jax.nn.sigmoid(g) * u).astype(jnp.bfloat16)
        y = lax.dot_general(act, d_ref[0].astype(jnp.bfloat16),
                            (((1,), (1,)), ((), ())),
                            preferred_element_type=jnp.float32)

        @pl.when(nf == 0)
        def _():
            o_ref[...] = y

        @pl.when((nf > 0) & (nf < NF - 1))
        def _():
            o_ref[...] += y

        @pl.when(nf == NF - 1)
        def _():
            o_ref[...] = (o_ref[...] + y) * wr_ref[:, 0:1]


def _run_mlp(te, xs, gate_proj, up_proj, down_proj, wrow):
    def _wmap(mt, nf, te, dim):
        cond = mt < te[120]
        return (jnp.where(cond, te[mt], E - 1),
                jnp.where(cond, nf, NF - 1), 0)[dim]

    return pl.pallas_call(
        _mlp_body,
        grid_spec=pltpu.PrefetchScalarGridSpec(
            num_scalar_prefetch=1,
            grid=(MT, NF),
            in_specs=[
                pl.BlockSpec(
                    (T, H // 2),
                    lambda mt, nf, te: (jnp.where(mt < te[120], mt,
                                                  te[120] - 1), 0)),
                pl.BlockSpec(
                    (1, FN, H),
                    lambda mt, nf, te: (_wmap(mt, nf, te, 0),
                                        _wmap(mt, nf, te, 1), 0)),
                pl.BlockSpec(
                    (1, FN, H),
                    lambda mt, nf, te: (_wmap(mt, nf, te, 0),
                                        _wmap(mt, nf, te, 1), 0)),
                pl.BlockSpec(
                    (1, H, FN),
                    lambda mt, nf, te: (_wmap(mt, nf, te, 0), 0,
                                        _wmap(mt, nf, te, 1))),
                pl.BlockSpec(
                    (T, WL),
                    lambda mt, nf, te: (jnp.where(mt < te[120], mt,
                                                  te[120] - 1), 0)),
            ],
            out_specs=pl.BlockSpec(
                (T, H),
                lambda mt, nf, te: (jnp.where(mt < te[120], mt,
                                              te[120] - 1), 0)),
        ),
        out_shape=jax.ShapeDtypeStruct((PAD_N, H), jnp.float32),
    )(te, xs, gate_proj, up_proj, down_proj, wrow)


@functools.cache
def _make_combine():
    mesh = plsc.VectorSubcoreMesh(core_axis_name="c", subcore_axis_name="s")

    @functools.partial(
        pl.kernel,
        out_type=jax.ShapeDtypeStruct((S, H), jnp.float32),
        mesh=mesh,
        scratch_types=[
            pltpu.VMEM((CHT, H), jnp.float32),
            pltpu.VMEM((CHT, H), jnp.float32),
            pltpu.VMEM((CHT,), jnp.int32),
            pltpu.VMEM((CHT,), jnp.int32),
            pltpu.SemaphoreType.DMA,
        ],
    )
    def combine(ys_hbm, pos0_hbm, pos1_hbm, out_hbm,
                r0_v, r1_v, i0_v, i1_v, sem):
        wid = lax.axis_index("s") * NC + lax.axis_index("c")
        for j in range(NCH):
            tb = wid * TOK_W + j * CHT
            pltpu.sync_copy(pos0_hbm.at[pl.ds(tb, CHT)], i0_v)
            pltpu.sync_copy(pos1_hbm.at[pl.ds(tb, CHT)], i1_v)
            c0 = pltpu.async_copy(ys_hbm.at[i0_v], r0_v, sem)
            c1 = pltpu.async_copy(ys_hbm.at[i1_v], r1_v, sem)
            c0.wait()
            c1.wait()
            for i in range(CHT):
                def _col(cc, _, i=i):
                    sl = pl.ds(cc * NL, NL)
                    r0_v[i, sl] = r0_v[i, sl] + r1_v[i, sl]
                    return 0

                lax.fori_loop(0, H // NL, _col, 0)
            pltpu.sync_copy(r0_v, out_hbm.at[pl.ds(tb, CHT)])

    return combine


def kernel(hidden_states, gate_w, gate_proj, up_proj, down_proj):
    b, s, h = hidden_states.shape
    x = hidden_states.reshape(s, h)

    logits, sel, topw, pos, te128, w0x, w1x = _run_router(x, gate_w)

    pos0 = pos[:, 0]
    pos1 = pos[:, 1]
    te = te128.reshape(128)

    hi = lax.bitcast_convert_type(x[:, :h // 2].astype(jnp.bfloat16),
                                  jnp.uint16).astype(jnp.int32)
    lo = lax.bitcast_convert_type(x[:, h // 2:].astype(jnp.bfloat16),
                                  jnp.uint16).astype(jnp.int32)
    xi = jnp.bitwise_or(jnp.left_shift(hi, 16), lo)
    xsi, wrow = _make_scatter_rows()(xi, pos0, pos1, w0x, w1x)
    ys = _run_mlp(te, xsi, gate_proj, up_proj, down_proj, wrow)
    final = _make_combine()(ys, pos0, pos1)

    return (final.reshape(b, s, h), sel.reshape(b, s, K),
            topw.reshape(b, s, K), logits.reshape(b, s, E))


# speedup vs baseline: 1.4554x; 1.3622x over previous
import functools

import jax
import jax.numpy as jnp
from jax import lax
from jax.experimental import pallas as pl
from jax.experimental.pallas import tpu as pltpu
from jax.experimental.pallas import tpu_sc as plsc

S = 2048
H = 2048
F = 4096
E = 8
K = 2

T = 576
MT = (S * K) // T + E
PAD_N = MT * T
FN = 512
NF = F // FN

NC, NS, NL = 2, 16, 16
NW = NC * NS
TOK_W = S // NW
CHT = 16
WL = 128
NCH = TOK_W // CHT


def _router_body(x_ref, gw_ref, logits_ref, sel_ref, topw_ref, pos_ref,
                 te_ref, w0x_ref, w1x_ref):
    x = x_ref[...]
    gw = gw_ref[...]
    logits = lax.dot_general(x, gw, (((1,), (1,)), ((), ())),
                             preferred_element_type=jnp.float32)
    logits_ref[...] = logits

    m = jnp.max(logits, axis=1, keepdims=True)
    p = jnp.exp(logits - m)
    rw = p / jnp.sum(p, axis=1, keepdims=True)

    iota_e = lax.broadcasted_iota(jnp.int32, (S, E), 1)
    w1 = jnp.max(rw, axis=1, keepdims=True)
    i1 = jnp.min(jnp.where(rw == w1, iota_e, E), axis=1, keepdims=True)
    rw2 = jnp.where(iota_e == i1, -jnp.inf, rw)
    w2 = jnp.max(rw2, axis=1, keepdims=True)
    i2 = jnp.min(jnp.where(rw2 == w2, iota_e, E), axis=1, keepdims=True)
    ssum = w1 + w2
    sel_ref[...] = jnp.concatenate([i1, i2], axis=1)
    topw_ref[...] = jnp.concatenate([w1 / ssum, w2 / ssum], axis=1)
    w0x_ref[...] = jnp.broadcast_to(w1 / ssum, (S, WL))
    w1x_ref[...] = jnp.broadcast_to(w2 / ssum, (S, WL))

    oh0 = (iota_e == i1).astype(jnp.float32)
    oh1 = (iota_e == i2).astype(jnp.float32)
    ohsum = oh0 + oh1
    CH = 256
    tri = (lax.broadcasted_iota(jnp.int32, (CH, CH), 0)
           > lax.broadcasted_iota(jnp.int32, (CH, CH), 1)).astype(jnp.float32)
    chunks = []
    carry = jnp.zeros((1, E), jnp.float32)
    for c in range(S // CH):
        blk = ohsum[c * CH:(c + 1) * CH, :]
        pref = lax.dot_general(tri, blk, (((1,), (0,)), ((), ())),
                               preferred_element_type=jnp.float32)
        chunks.append(pref + carry)
        carry = carry + jnp.sum(blk, axis=0, keepdims=True)
    P = jnp.concatenate(chunks, axis=0)
    counts = carry

    rank0 = jnp.sum(P * oh0, axis=1, keepdims=True)
    rank1 = jnp.sum(P * oh1, axis=1, keepdims=True)

    cap = jnp.ceil(counts / T) * T
    upper = (lax.broadcasted_iota(jnp.int32, (E, E), 0)
             < lax.broadcasted_iota(jnp.int32, (E, E), 1)).astype(jnp.float32)
    start = lax.dot_general(cap, upper, (((1,), (0,)), ((), ())),
                            preferred_element_type=jnp.float32)
    ends = start + cap

    pos0 = rank0 + jnp.sum(oh0 * start, axis=1, keepdims=True)
    pos1 = rank1 + jnp.sum(oh1 * start, axis=1, keepdims=True)
    pos_ref[...] = jnp.concatenate([pos0, pos1], axis=1).astype(jnp.int32)

    tile_base = lax.broadcasted_iota(jnp.int32, (1, 128), 1) * T
    te = jnp.zeros((1, 128), jnp.int32)
    for e in range(E):
        end_e = ends[0, e].astype(jnp.int32)
        te = te + (tile_base >= end_e).astype(jnp.int32)
    te = jnp.minimum(te, E - 1)
    used = (ends[0, E - 1] / T).astype(jnp.int32)
    lane = lax.broadcasted_iota(jnp.int32, (1, 128), 1)
    te_ref[...] = jnp.where(lane == 120, used, te)


def _run_router(x, gate_w):
    return pl.pallas_call(
        _router_body,
        out_shape=[
            jax.ShapeDtypeStruct((S, E), jnp.float32),
            jax.ShapeDtypeStruct((S, K), jnp.int32),
            jax.ShapeDtypeStruct((S, K), jnp.float32),
            jax.ShapeDtypeStruct((S, K), jnp.int32),
            jax.ShapeDtypeStruct((1, 128), jnp.int32),
            jax.ShapeDtypeStruct((S, WL), jnp.float32),
            jax.ShapeDtypeStruct((S, WL), jnp.float32),
        ],
    )(x, gate_w)


@functools.cache
def _make_scatter_rows():
    mesh = plsc.VectorSubcoreMesh(core_axis_name="c", subcore_axis_name="s")

    @functools.partial(
        pl.kernel,
        out_type=[
            jax.ShapeDtypeStruct((PAD_N, H // 2), jnp.int32),
            jax.ShapeDtypeStruct((PAD_N, WL), jnp.float32),
        ],
        mesh=mesh,
        scratch_types=[
            pltpu.VMEM((CHT, H // 2), jnp.int32),
            pltpu.VMEM((CHT,), jnp.int32),
            pltpu.VMEM((CHT,), jnp.int32),
            pltpu.VMEM((CHT, WL), jnp.float32),
            pltpu.VMEM((CHT, WL), jnp.float32),
            pltpu.SemaphoreType.DMA,
        ],
    )
    def scatter_rows(x_hbm, pos0_hbm, pos1_hbm, w0x_hbm, w1x_hbm,
                     xs_hbm, wrow_hbm, rows_v, i0_v, i1_v, w0m_v, w1m_v,
                     sem):
        wid = lax.axis_index("s") * NC + lax.axis_index("c")
        for j in range(NCH):
            tb = wid * TOK_W + j * CHT
            pltpu.sync_copy(x_hbm.at[pl.ds(tb, CHT)], rows_v)
            pltpu.sync_copy(pos0_hbm.at[pl.ds(tb, CHT)], i0_v)
            pltpu.sync_copy(pos1_hbm.at[pl.ds(tb, CHT)], i1_v)
            pltpu.sync_copy(w0x_hbm.at[pl.ds(tb, CHT)], w0m_v)
            pltpu.sync_copy(w1x_hbm.at[pl.ds(tb, CHT)], w1m_v)
            pltpu.async_copy(rows_v, xs_hbm.at[i0_v], sem).wait()
            pltpu.async_copy(rows_v, xs_hbm.at[i1_v], sem).wait()
            pltpu.async_copy(w0m_v, wrow_hbm.at[i0_v], sem).wait()
            pltpu.async_copy(w1m_v, wrow_hbm.at[i1_v], sem).wait()

    return scatter_rows


def _mlp_body(te_ref, xs_ref, g_ref, u_ref, d_ref, wr_ref, o_ref):
    mt = pl.program_id(0)
    nf = pl.program_id(1)

    @pl.when(mt < te_ref[120])
    def _():
        xi = xs_ref[...]
        xa = lax.bitcast_convert_type(
            jnp.bitwise_and(xi, jnp.int32(-65536)), jnp.float32)
        xb = lax.bitcast_convert_type(jnp.left_shift(xi, 16), jnp.float32)
        x = jnp.concatenate([xa, xb], axis=1).astype(jnp.bfloat16)
        g = lax.dot_general(x, g_ref[0].astype(jnp.bfloat16),
                            (((1,), (1,)), ((), ())),
                            preferred_element_type=jnp.float32)
        u = lax.dot_general(x, u_ref[0].astype(jnp.bfloat16),
                            (((1,), (1,)), ((), ())),
                            preferred_element_type=jnp.float32)
        act = (g * jax.nn.sigmoid(g) * u).astype(jnp.bfloat16)
        y = lax.dot_general(act, d_ref[0].astype(jnp.bfloat16),
                            (((1,), (1,)), ((), ())),
                            preferred_element_type=jnp.float32)

        @pl.when(nf == 0)
        def _():
            o_ref[...] = y

        @pl.when((nf > 0) & (nf < NF - 1))
        def _():
            o_ref[...] += y

        @pl.when(nf == NF - 1)
        def _():
            o_ref[...] = (o_ref[...] + y) * wr_ref[:, 0:1]


def _run_mlp(te, xs, gate_proj, up_proj, down_proj, wrow):
    def _wmap(mt, nf, te, dim):
        cond = mt < te[120]
        return (jnp.where(cond, te[mt], E - 1),
                jnp.where(cond, nf, NF - 1), 0)[dim]

    return pl.pallas_call(
        _mlp_body,
        grid_spec=pltpu.PrefetchScalarGridSpec(
            num_scalar_prefetch=1,
            grid=(MT, NF),
            in_specs=[
                pl.BlockSpec(
                    (T, H // 2),
                    lambda mt, nf, te: (jnp.where(mt < te[120], mt,
                                                  te[120] - 1), 0)),
                pl.BlockSpec(
                    (1, FN, H),
                    lambda mt, nf, te: (_wmap(mt, nf, te, 0),
                                        _wmap(mt, nf, te, 1), 0)),
                pl.BlockSpec(
                    (1, FN, H),
                    lambda mt, nf, te: (_wmap(mt, nf, te, 0),
                                        _wmap(mt, nf, te, 1), 0)),
                pl.BlockSpec(
                    (1, H, FN),
                    lambda mt, nf, te: (_wmap(mt, nf, te, 0), 0,
                                        _wmap(mt, nf, te, 1))),
                pl.BlockSpec(
                    (T, WL),
                    lambda mt, nf, te: (jnp.where(mt < te[120], mt,
                                                  te[120] - 1), 0)),
            ],
            out_specs=pl.BlockSpec(
                (T, H),
                lambda mt, nf, te: (jnp.where(mt < te[120], mt,
                                              te[120] - 1), 0)),
        ),
        out_shape=jax.ShapeDtypeStruct((PAD_N, H), jnp.float32),
    )(te, xs, gate_proj, up_proj, down_proj, wrow)


@functools.cache
def _make_combine():
    mesh = plsc.VectorSubcoreMesh(core_axis_name="c", subcore_axis_name="s")

    @functools.partial(
        pl.kernel,
        out_type=jax.ShapeDtypeStruct((S, H), jnp.float32),
        mesh=mesh,
        scratch_types=[
            pltpu.VMEM((CHT, H), jnp.float32),
            pltpu.VMEM((CHT, H), jnp.float32),
            pltpu.VMEM((CHT,), jnp.int32),
            pltpu.VMEM((CHT,), jnp.int32),
            pltpu.SemaphoreType.DMA,
        ],
    )
    def combine(ys_hbm, pos0_hbm, pos1_hbm, out_hbm,
                r0_v, r1_v, i0_v, i1_v, sem):
        wid = lax.axis_index("s") * NC + lax.axis_index("c")
        for j in range(NCH):
            tb = wid * TOK_W + j * CHT
            pltpu.sync_copy(pos0_hbm.at[pl.ds(tb, CHT)], i0_v)
            pltpu.sync_copy(pos1_hbm.at[pl.ds(tb, CHT)], i1_v)
            c0 = pltpu.async_copy(ys_hbm.at[i0_v], r0_v, sem)
            c1 = pltpu.async_copy(ys_hbm.at[i1_v], r1_v, sem)
            c0.wait()
            c1.wait()
            for i in range(CHT):
                def _col(cc, _, i=i):
                    sl = pl.ds(cc * NL, NL)
                    r0_v[i, sl] = r0_v[i, sl] + r1_v[i, sl]
                    return 0

                lax.fori_loop(0, H // NL, _col, 0)
            pltpu.sync_copy(r0_v, out_hbm.at[pl.ds(tb, CHT)])

    return combine


def kernel(hidden_states, gate_w, gate_proj, up_proj, down_proj):
    b, s, h = hidden_states.shape
    x = hidden_states.reshape(s, h)

    logits, sel, topw, pos, te128, w0x, w1x = _run_router(x, gate_w)

    pos0 = pos[:, 0]
    pos1 = pos[:, 1]
    te = te128.reshape(128)

    hi = lax.bitcast_convert_type(x[:, :h // 2].astype(jnp.bfloat16),
                                  jnp.uint16).astype(jnp.int32)
    lo = lax.bitcast_convert_type(x[:, h // 2:].astype(jnp.bfloat16),
                                  jnp.uint16).astype(jnp.int32)
    xi = jnp.bitwise_or(jnp.left_shift(hi, 16), lo)
    xsi, wrow = _make_scatter_rows()(xi, pos0, pos1, w0x, w1x)
    ys = _run_mlp(te, xsi, gate_proj, up_proj, down_proj, wrow)
    final = _make_combine()(ys, pos0, pos1)

    return (final.reshape(b, s, h), sel.reshape(b, s, K),
            topw.reshape(b, s, K), logits.reshape(b, s, E))
